# baseline (speedup 1.0000x reference)
import functools

import jax
import jax.numpy as jnp
from jax.experimental import pallas as pl
from jax.experimental.pallas import tpu as pltpu


def _pipe_kernel(x_hbm, w1_ref, b1_ref, w2_ref, b2_ref, o_hbm,
                 x_buf, o_buf, in_sem, out_sem, *, bt, n_steps):
    def dma_in(slot, step):
        return pltpu.make_async_copy(
            x_hbm.at[pl.ds(step * bt, bt), :], x_buf.at[slot],
            in_sem.at[slot])

    def dma_out(slot, step):
        return pltpu.make_async_copy(
            o_buf.at[slot], o_hbm.at[pl.ds(step * bt, bt), :],
            out_sem.at[slot])

    dma_in(0, 0).start()

    def body(step, _):
        cur = jax.lax.rem(step, 2)
        nxt = jax.lax.rem(step + 1, 2)

        @pl.when(step + 1 < n_steps)
        def _():
            dma_in(nxt, step + 1).start()

        dma_in(cur, step).wait()

        @pl.when(step >= 2)
        def _():
            dma_out(cur, step - 2).wait()

        h = jnp.dot(x_buf[cur], w1_ref[...],
                    preferred_element_type=jnp.float32)
        h = jnp.maximum(h + b1_ref[...], 0.0)
        out = jnp.dot(h, w2_ref[...], preferred_element_type=jnp.float32)
        o_buf[cur] = (out + b2_ref[...]).astype(o_buf.dtype)

        dma_out(cur, step).start()
        return ()

    jax.lax.fori_loop(0, n_steps, body, (), unroll=False)

    if n_steps >= 2:
        dma_out((n_steps - 2) % 2, n_steps - 2).wait()
    dma_out((n_steps - 1) % 2, n_steps - 1).wait()


def _round_up(n, m):
    return ((n + m - 1) // m) * m


def kernel(x, w1, b1, w2, b2):
    B, A = x.shape
    H = w1.shape[1]
    O = w2.shape[1]

    Ap = max(_round_up(A, 128), 128)
    Hp = max(_round_up(H, 128), 128)
    Op = max(_round_up(O, 128), 128)

    bt = 2048
    Bg = max(_round_up(B, bt), bt)
    n_steps = Bg // bt

    xp = x
    if (Bg, Ap) != (B, A):
        xp = jnp.zeros((Bg, Ap), x.dtype).at[:B, :A].set(x)
    w1p = w1
    if (Ap, Hp) != w1.shape:
        w1p = jnp.zeros((Ap, Hp), w1.dtype).at[:A, :H].set(w1)
    w2p = w2
    if (Hp, Op) != w2.shape:
        w2p = jnp.zeros((Hp, Op), w2.dtype).at[:H, :O].set(w2)
    b1p = jnp.zeros((1, Hp), b1.dtype).at[0, :H].set(b1)
    b2p = jnp.zeros((1, Op), b2.dtype).at[0, :O].set(b2)

    flops = 2 * Bg * Ap * Hp + 2 * Bg * Hp * Op
    bytes_accessed = 4 * (Bg * Ap + Ap * Hp + Hp + Hp * Op + Op + Bg * Op)

    vmem = functools.partial(pl.BlockSpec, memory_space=pltpu.MemorySpace.VMEM)
    outp = pl.pallas_call(
        functools.partial(_pipe_kernel, bt=bt, n_steps=n_steps),
        out_shape=jax.ShapeDtypeStruct((Bg, Op), x.dtype),
        in_specs=[
            pl.BlockSpec(memory_space=pltpu.MemorySpace.HBM),
            vmem((Ap, Hp), lambda: (0, 0)),
            vmem((1, Hp), lambda: (0, 0)),
            vmem((Hp, Op), lambda: (0, 0)),
            vmem((1, Op), lambda: (0, 0)),
        ],
        out_specs=pl.BlockSpec(memory_space=pltpu.MemorySpace.HBM),
        scratch_shapes=[
            pltpu.VMEM((2, bt, Ap), x.dtype),
            pltpu.VMEM((2, bt, Op), x.dtype),
            pltpu.SemaphoreType.DMA((2,)),
            pltpu.SemaphoreType.DMA((2,)),
        ],
        compiler_params=pltpu.CompilerParams(
            vmem_limit_bytes=100 * 1024 * 1024,
        ),
        cost_estimate=pl.CostEstimate(
            flops=flops, transcendentals=0, bytes_accessed=bytes_accessed),
    )(xp, w1p, b1p, w2p, b2p)

    if (Bg, Op) != (B, O):
        outp = outp[:B, :O]
    return outp


# speedup vs baseline: 1.0979x; 1.0979x over previous
import jax
import jax.numpy as jnp
from jax.experimental import pallas as pl
from jax.experimental.pallas import tpu as pltpu


def _mlp_kernel(x_ref, w1_ref, b1_ref, w2_ref, b2_ref, o_ref):
    xb = x_ref[...].astype(jnp.bfloat16)
    h = jnp.dot(xb, w1_ref[...], preferred_element_type=jnp.float32)
    h = jnp.maximum(h + b1_ref[...], 0.0).astype(jnp.bfloat16)
    out = jnp.dot(h, w2_ref[...], preferred_element_type=jnp.float32)
    o_ref[...] = (out + b2_ref[...]).astype(o_ref.dtype)


def _round_up(n, m):
    return ((n + m - 1) // m) * m


def kernel(x, w1, b1, w2, b2):
    B, A = x.shape
    H = w1.shape[1]
    O = w2.shape[1]

    Ap = max(_round_up(A, 128), 128)
    Hp = max(_round_up(H, 128), 128)
    Op = max(_round_up(O, 128), 128)

    bt = 8192
    Bg = max(_round_up(B, bt), bt)

    xp = x
    if (Bg, Ap) != (B, A):
        xp = jnp.zeros((Bg, Ap), x.dtype).at[:B, :A].set(x)
    w1p = jnp.zeros((Ap, Hp), jnp.bfloat16).at[:A, :H].set(
        w1.astype(jnp.bfloat16))
    w2p = jnp.zeros((Hp, Op), jnp.bfloat16).at[:H, :O].set(
        w2.astype(jnp.bfloat16))
    b1p = jnp.zeros((1, Hp), b1.dtype).at[0, :H].set(b1)
    b2p = jnp.zeros((1, Op), b2.dtype).at[0, :O].set(b2)

    flops = 2 * Bg * Ap * Hp + 2 * Bg * Hp * Op
    bytes_accessed = (4 * (Bg * Ap + Bg * Op) + 2 * (Ap * Hp + Hp * Op)
                      + 4 * (Hp + Op))

    outp = pl.pallas_call(
        _mlp_kernel,
        out_shape=jax.ShapeDtypeStruct((Bg, Op), x.dtype),
        grid=(Bg // bt,),
        in_specs=[
            pl.BlockSpec((bt, Ap), lambda i: (i, 0)),
            pl.BlockSpec((Ap, Hp), lambda i: (0, 0)),
            pl.BlockSpec((1, Hp), lambda i: (0, 0)),
            pl.BlockSpec((Hp, Op), lambda i: (0, 0)),
            pl.BlockSpec((1, Op), lambda i: (0, 0)),
        ],
        out_specs=pl.BlockSpec((bt, Op), lambda i: (i, 0)),
        compiler_params=pltpu.CompilerParams(
            dimension_semantics=("parallel",),
            vmem_limit_bytes=100 * 1024 * 1024,
        ),
        cost_estimate=pl.CostEstimate(
            flops=flops, transcendentals=0, bytes_accessed=bytes_accessed),
    )(xp, w1p, b1p, w2p, b2p)

    if (Bg, Op) != (B, O):
        outp = outp[:B, :O]
    return outp


# speedup vs baseline: 1.2568x; 1.1448x over previous
import functools

import jax
import jax.numpy as jnp
from jax.experimental import pallas as pl
from jax.experimental.pallas import tpu as pltpu

_NSLOTS = 4


def _pipe_kernel(x_hbm, w1_ref, b1_ref, w2_ref, b2_ref, o_hbm,
                 x_buf, o_buf, in_sem, out_sem, *, bt, n_steps):
    def dma_in(slot, step):
        return pltpu.make_async_copy(
            x_hbm.at[pl.ds(step * bt, bt), :], x_buf.at[slot],
            in_sem.at[slot])

    def dma_out(slot, step):
        return pltpu.make_async_copy(
            o_buf.at[slot], o_hbm.at[pl.ds(step * bt, bt), :],
            out_sem.at[slot])

    for s in range(min(_NSLOTS - 1, n_steps)):
        dma_in(s, s).start()

    def body(step, _):
        cur = jax.lax.rem(step, _NSLOTS)
        ocur = jax.lax.rem(step, 2)

        @pl.when(step + _NSLOTS - 1 < n_steps)
        def _():
            dma_in(jax.lax.rem(step + _NSLOTS - 1, _NSLOTS),
                   step + _NSLOTS - 1).start()

        dma_in(cur, step).wait()

        @pl.when(step >= 2)
        def _():
            dma_out(ocur, step - 2).wait()

        h = jnp.dot(x_buf[cur], w1_ref[...],
                    preferred_element_type=jnp.float32)
        h = jnp.maximum(h + b1_ref[...], 0.0)
        out = jnp.dot(h, w2_ref[...], preferred_element_type=jnp.float32)
        o_buf[ocur] = (out + b2_ref[...]).astype(o_buf.dtype)

        dma_out(ocur, step).start()
        return ()

    jax.lax.fori_loop(0, n_steps, body, (), unroll=False)

    if n_steps >= 2:
        dma_out((n_steps - 2) % 2, n_steps - 2).wait()
    dma_out((n_steps - 1) % 2, n_steps - 1).wait()


def _round_up(n, m):
    return ((n + m - 1) // m) * m


def kernel(x, w1, b1, w2, b2):
    B, A = x.shape
    H = w1.shape[1]
    O = w2.shape[1]

    Ap = max(_round_up(A, 128), 128)
    Hp = max(_round_up(H, 128), 128)
    Op = max(_round_up(O, 128), 128)

    bt = 4096
    Bg = max(_round_up(B, bt), bt)
    n_steps = Bg // bt

    xp = x
    if (Bg, Ap) != (B, A):
        xp = jnp.zeros((Bg, Ap), x.dtype).at[:B, :A].set(x)
    w1p = w1
    if (Ap, Hp) != w1.shape:
        w1p = jnp.zeros((Ap, Hp), w1.dtype).at[:A, :H].set(w1)
    w2p = w2
    if (Hp, Op) != w2.shape:
        w2p = jnp.zeros((Hp, Op), w2.dtype).at[:H, :O].set(w2)
    b1p = jnp.zeros((1, Hp), b1.dtype).at[0, :H].set(b1)
    b2p = jnp.zeros((1, Op), b2.dtype).at[0, :O].set(b2)

    flops = 2 * Bg * Ap * Hp + 2 * Bg * Hp * Op
    bytes_accessed = 4 * (Bg * Ap + Ap * Hp + Hp + Hp * Op + Op + Bg * Op)

    vmem = functools.partial(pl.BlockSpec, memory_space=pltpu.MemorySpace.VMEM)
    outp = pl.pallas_call(
        functools.partial(_pipe_kernel, bt=bt, n_steps=n_steps),
        out_shape=jax.ShapeDtypeStruct((Bg, Op), x.dtype),
        in_specs=[
            pl.BlockSpec(memory_space=pltpu.MemorySpace.HBM),
            vmem((Ap, Hp), lambda: (0, 0)),
            vmem((1, Hp), lambda: (0, 0)),
            vmem((Hp, Op), lambda: (0, 0)),
            vmem((1, Op), lambda: (0, 0)),
        ],
        out_specs=pl.BlockSpec(memory_space=pltpu.MemorySpace.HBM),
        scratch_shapes=[
            pltpu.VMEM((_NSLOTS, bt, Ap), x.dtype),
            pltpu.VMEM((2, bt, Op), x.dtype),
            pltpu.SemaphoreType.DMA((_NSLOTS,)),
            pltpu.SemaphoreType.DMA((2,)),
        ],
        compiler_params=pltpu.CompilerParams(
            vmem_limit_bytes=100 * 1024 * 1024,
        ),
        cost_estimate=pl.CostEstimate(
            flops=flops, transcendentals=0, bytes_accessed=bytes_accessed),
    )(xp, w1p, b1p, w2p, b2p)

    if (Bg, Op) != (B, O):
        outp = outp[:B, :O]
    return outp
